# baseline (speedup 1.0000x reference)
import functools

import jax
import jax.numpy as jnp
from jax import lax
from jax.experimental import pallas as pl
from jax.experimental.pallas import tpu as pltpu
from jax.experimental.pallas import tpu_sc as plsc

NN = 10000
DIM = 128
KTOP = 32
ALPHA = 3.0
ROWS = 200
ONE_BITS = 0x3F800000
WIN = 256
WIN_BITS = 8


SC_B = 10240
SC_PER = SC_B // 32
SC_CHUNKS = (128, 128, 64)


def _sc_gather_body(x_hbm, t_hbm, o_hbm, idx_v, r_v, sem):
    wid = lax.axis_index("s") * 2 + lax.axis_index("c")
    base = wid * SC_PER
    pltpu.sync_copy(x_hbm.at[pl.ds(base, SC_PER)], idx_v)
    copies = []
    off = 0
    for w in SC_CHUNKS:
        sl = pl.ds(off, w)
        copies.append(pltpu.async_copy(t_hbm.at[idx_v.at[sl]], r_v.at[sl],
                                       sem))
        off += w
    for c in copies:
        c.wait()
    pltpu.sync_copy(r_v, o_hbm.at[pl.ds(base, SC_PER)])


@functools.partial(
    pl.kernel,
    mesh=plsc.VectorSubcoreMesh(core_axis_name="c", subcore_axis_name="s"),
    out_type=jax.ShapeDtypeStruct((SC_B, DIM), jnp.float32),
    scratch_types=[pltpu.VMEM((SC_PER,), jnp.int32),
                   pltpu.VMEM((SC_PER, DIM), jnp.float32),
                   pltpu.SemaphoreType.DMA],
)
def _sc_gather(*refs):
    _sc_gather_body(*refs)


def _gather_tables(x, emb1, emb2):
    xp = jnp.concatenate([x, jnp.zeros((SC_B - NN,), jnp.int32)])
    return _sc_gather(xp, emb1), _sc_gather(xp, emb2)


def _body(e1_ref, e2_ref, w1_ref, b1_ref, w2_ref, b2_ref, out_ref, nv2_ref):
    @pl.when(pl.program_id(0) == 0)
    def _():
        z = lax.dot_general(e2_ref[...], w2_ref[...], (((1,), (1,)), ((), ())),
                            preferred_element_type=jnp.float32)
        nv2_ref[...] = jnp.tanh(ALPHA * (z + b2_ref[...]))

    h = lax.dot_general(e1_ref[...], w1_ref[...], (((1,), (1,)), ((), ())),
                        preferred_element_type=jnp.float32)
    nv1 = jnp.tanh(ALPHA * (h + b1_ref[...]))

    aw = lax.dot_general(nv1, nv2_ref[:WIN, :], (((1,), (1,)), ((), ())),
                         preferred_element_type=jnp.float32)
    bw = lax.bitcast_convert_type(jnp.maximum(jnp.tanh(ALPHA * aw), 0.0),
                                  jnp.int32)
    colw = lax.broadcasted_iota(jnp.int32, (ROWS, WIN), 1)
    ecw = jnp.where(bw == ONE_BITS, colw, jnp.int32(0x7FFFFFFF))
    cntw = jnp.sum((ecw < WIN).astype(jnp.int32), axis=1, keepdims=True)
    fast = jnp.min(cntw) >= KTOP

    @pl.when(fast)
    def _fast():
        def bs(_, lohi):
            lo, hi = lohi
            mid = lo + ((hi - lo) >> 1)
            cnt = jnp.sum((ecw < mid).astype(jnp.int32), axis=1,
                          keepdims=True)
            ge = cnt >= KTOP
            return jnp.where(ge, lo, mid), jnp.where(ge, mid, hi)

        lo1 = jnp.zeros((ROWS, 1), jnp.int32)
        hi1 = jnp.full((ROWS, 1), WIN, jnp.int32)
        _, p = lax.fori_loop(0, WIN_BITS, bs, (lo1, hi1))
        out_ref[:, :WIN] = (ecw < p).astype(jnp.float32)
        out_ref[:, WIN:] = jnp.zeros((ROWS, NN - WIN), jnp.float32)

    @pl.when(jnp.logical_not(fast))
    def _slow():
        a = lax.dot_general(nv1, nv2_ref[...], (((1,), (1,)), ((), ())),
                            preferred_element_type=jnp.float32)
        adj = jnp.maximum(jnp.tanh(ALPHA * a), 0.0)
        bi = lax.bitcast_convert_type(adj, jnp.int32)

        def bs_val(_, lohi):
            lo, hi = lohi
            mid = lo + ((hi - lo) >> 1)
            cnt = jnp.sum((bi >= mid).astype(jnp.int32), axis=1,
                          keepdims=True)
            ge = cnt >= KTOP
            return jnp.where(ge, mid, lo), jnp.where(ge, hi, mid)

        lo0 = jnp.zeros((ROWS, 1), jnp.int32)
        hi0 = jnp.full((ROWS, 1), ONE_BITS + 1, jnp.int32)
        vk, _ = lax.fori_loop(0, 31, bs_val, (lo0, hi0))
        cgt = jnp.sum((bi > vk).astype(jnp.int32), axis=1, keepdims=True)
        m = KTOP - cgt

        colid = lax.broadcasted_iota(jnp.int32, (ROWS, NN), 1)
        ec = jnp.where(bi == vk, colid, jnp.int32(0x7FFFFFFF))

        def bs_idx(_, lohi):
            lo, hi = lohi
            mid = lo + ((hi - lo) >> 1)
            cnt = jnp.sum((ec < mid).astype(jnp.int32), axis=1,
                          keepdims=True)
            ge = cnt >= m
            return jnp.where(ge, lo, mid), jnp.where(ge, mid, hi)

        lo1 = jnp.zeros((ROWS, 1), jnp.int32)
        hi1 = jnp.full((ROWS, 1), 16384, jnp.int32)
        _, p = lax.fori_loop(0, 14, bs_idx, (lo1, hi1))

        keep = (bi > vk) | (ec < p)
        out_ref[...] = jnp.where(keep, adj, 0.0)


def kernel(x, emb1, emb2, W1, b1, W2, b2):
    e1, e2 = _gather_tables(x, emb1, emb2)
    return pl.pallas_call(
        _body,
        grid=(NN // ROWS,),
        in_specs=[
            pl.BlockSpec((ROWS, DIM), lambda i: (i, 0)),
            pl.BlockSpec((NN, DIM), lambda i: (0, 0)),
            pl.BlockSpec((DIM, DIM), lambda i: (0, 0)),
            pl.BlockSpec((1, DIM), lambda i: (0, 0)),
            pl.BlockSpec((DIM, DIM), lambda i: (0, 0)),
            pl.BlockSpec((1, DIM), lambda i: (0, 0)),
        ],
        out_specs=pl.BlockSpec((ROWS, NN), lambda i: (i, 0)),
        out_shape=jax.ShapeDtypeStruct((NN, NN), jnp.float32),
        scratch_shapes=[pltpu.VMEM((NN, DIM), jnp.float32)],
    )(e1, e2, W1, b1.reshape(1, DIM), W2, b2.reshape(1, DIM))


# speedup vs baseline: 1.0819x; 1.0819x over previous
import functools

import jax
import jax.numpy as jnp
from jax import lax
from jax.experimental import pallas as pl
from jax.experimental.pallas import tpu as pltpu
from jax.experimental.pallas import tpu_sc as plsc

NN = 10000
DIM = 128
KTOP = 32
ALPHA = 3.0
ROWS = 200
ONE_BITS = 0x3F800000
WIN = 256
WIN_BITS = 8


SC_B = 10240
SC_PER = SC_B // 32
SC_CHUNKS = (128, 128, 64)


def _sc_gather_body(x_hbm, t1_hbm, t2_hbm, o1_hbm, o2_hbm,
                    idx_v, r1_v, r2_v, sem):
    wid = lax.axis_index("s") * 2 + lax.axis_index("c")
    base = wid * SC_PER
    pltpu.sync_copy(x_hbm.at[pl.ds(base, SC_PER)], idx_v)
    copies = []
    off = 0
    for w in SC_CHUNKS:
        sl = pl.ds(off, w)
        copies.append(pltpu.async_copy(t1_hbm.at[idx_v.at[sl]], r1_v.at[sl],
                                       sem))
        copies.append(pltpu.async_copy(t2_hbm.at[idx_v.at[sl]], r2_v.at[sl],
                                       sem))
        off += w
    for c in copies:
        c.wait()
    o1 = pltpu.async_copy(r1_v, o1_hbm.at[pl.ds(base, SC_PER)], sem)
    o2 = pltpu.async_copy(r2_v, o2_hbm.at[pl.ds(base, SC_PER)], sem)
    o1.wait()
    o2.wait()


@functools.partial(
    pl.kernel,
    mesh=plsc.VectorSubcoreMesh(core_axis_name="c", subcore_axis_name="s"),
    out_type=[jax.ShapeDtypeStruct((SC_B, DIM), jnp.float32),
              jax.ShapeDtypeStruct((SC_B, DIM), jnp.float32)],
    scratch_types=[pltpu.VMEM((SC_PER,), jnp.int32),
                   pltpu.VMEM((SC_PER, DIM), jnp.float32),
                   pltpu.VMEM((SC_PER, DIM), jnp.float32),
                   pltpu.SemaphoreType.DMA],
)
def _sc_gather(*refs):
    _sc_gather_body(*refs)


def _gather_tables(x, emb1, emb2):
    xp = jnp.concatenate([x, jnp.zeros((SC_B - NN,), jnp.int32)])
    return _sc_gather(xp, emb1, emb2)


def _body(e1_ref, e2_ref, w1_ref, b1_ref, w2_ref, b2_ref, out_ref, nv2_ref):
    @pl.when(pl.program_id(0) == 0)
    def _():
        z = lax.dot_general(e2_ref[...], w2_ref[...], (((1,), (1,)), ((), ())),
                            preferred_element_type=jnp.float32)
        nv2_ref[...] = jnp.tanh(ALPHA * (z + b2_ref[...]))

    h = lax.dot_general(e1_ref[...], w1_ref[...], (((1,), (1,)), ((), ())),
                        preferred_element_type=jnp.float32)
    nv1 = jnp.tanh(ALPHA * (h + b1_ref[...]))

    aw = lax.dot_general(nv1, nv2_ref[:WIN, :], (((1,), (1,)), ((), ())),
                         preferred_element_type=jnp.float32)
    bw = lax.bitcast_convert_type(jnp.maximum(jnp.tanh(ALPHA * aw), 0.0),
                                  jnp.int32)
    colw = lax.broadcasted_iota(jnp.int32, (ROWS, WIN), 1)
    ecw = jnp.where(bw == ONE_BITS, colw, jnp.int32(0x7FFFFFFF))
    cntw = jnp.sum((ecw < WIN).astype(jnp.int32), axis=1, keepdims=True)
    fast = jnp.min(cntw) >= KTOP

    @pl.when(fast)
    def _fast():
        def bs(_, lohi):
            lo, hi = lohi
            mid = lo + ((hi - lo) >> 1)
            cnt = jnp.sum((ecw < mid).astype(jnp.int32), axis=1,
                          keepdims=True)
            ge = cnt >= KTOP
            return jnp.where(ge, lo, mid), jnp.where(ge, mid, hi)

        lo1 = jnp.zeros((ROWS, 1), jnp.int32)
        hi1 = jnp.full((ROWS, 1), WIN, jnp.int32)
        _, p = lax.fori_loop(0, WIN_BITS, bs, (lo1, hi1))
        out_ref[:, :WIN] = (ecw < p).astype(jnp.float32)
        out_ref[:, WIN:] = jnp.zeros((ROWS, NN - WIN), jnp.float32)

    @pl.when(jnp.logical_not(fast))
    def _slow():
        a = lax.dot_general(nv1, nv2_ref[...], (((1,), (1,)), ((), ())),
                            preferred_element_type=jnp.float32)
        adj = jnp.maximum(jnp.tanh(ALPHA * a), 0.0)
        bi = lax.bitcast_convert_type(adj, jnp.int32)

        def bs_val(_, lohi):
            lo, hi = lohi
            mid = lo + ((hi - lo) >> 1)
            cnt = jnp.sum((bi >= mid).astype(jnp.int32), axis=1,
                          keepdims=True)
            ge = cnt >= KTOP
            return jnp.where(ge, mid, lo), jnp.where(ge, hi, mid)

        lo0 = jnp.zeros((ROWS, 1), jnp.int32)
        hi0 = jnp.full((ROWS, 1), ONE_BITS + 1, jnp.int32)
        vk, _ = lax.fori_loop(0, 31, bs_val, (lo0, hi0))
        cgt = jnp.sum((bi > vk).astype(jnp.int32), axis=1, keepdims=True)
        m = KTOP - cgt

        colid = lax.broadcasted_iota(jnp.int32, (ROWS, NN), 1)
        ec = jnp.where(bi == vk, colid, jnp.int32(0x7FFFFFFF))

        def bs_idx(_, lohi):
            lo, hi = lohi
            mid = lo + ((hi - lo) >> 1)
            cnt = jnp.sum((ec < mid).astype(jnp.int32), axis=1,
                          keepdims=True)
            ge = cnt >= m
            return jnp.where(ge, lo, mid), jnp.where(ge, mid, hi)

        lo1 = jnp.zeros((ROWS, 1), jnp.int32)
        hi1 = jnp.full((ROWS, 1), 16384, jnp.int32)
        _, p = lax.fori_loop(0, 14, bs_idx, (lo1, hi1))

        keep = (bi > vk) | (ec < p)
        out_ref[...] = jnp.where(keep, adj, 0.0)


def kernel(x, emb1, emb2, W1, b1, W2, b2):
    e1, e2 = _gather_tables(x, emb1, emb2)
    return pl.pallas_call(
        _body,
        grid=(NN // ROWS,),
        in_specs=[
            pl.BlockSpec((ROWS, DIM), lambda i: (i, 0)),
            pl.BlockSpec((NN, DIM), lambda i: (0, 0)),
            pl.BlockSpec((DIM, DIM), lambda i: (0, 0)),
            pl.BlockSpec((1, DIM), lambda i: (0, 0)),
            pl.BlockSpec((DIM, DIM), lambda i: (0, 0)),
            pl.BlockSpec((1, DIM), lambda i: (0, 0)),
        ],
        out_specs=pl.BlockSpec((ROWS, NN), lambda i: (i, 0)),
        out_shape=jax.ShapeDtypeStruct((NN, NN), jnp.float32),
        scratch_shapes=[pltpu.VMEM((NN, DIM), jnp.float32)],
    )(e1, e2, W1, b1.reshape(1, DIM), W2, b2.reshape(1, DIM))
